# Initial kernel scaffold; baseline (speedup 1.0000x reference)
import jax
import jax.numpy as jnp
from jax.experimental import pallas as pl


def kernel(x, edge_index, global_features, batch, W1, b1, W2, b2, fW1, fb1, fW2, fb2):
    raise NotImplementedError("write your pallas kernel here")


# speedup vs baseline: 21.0405x; 21.0405x over previous
import functools

import jax
import jax.numpy as jnp
from jax import lax
from jax.experimental import pallas as pl
from jax.experimental.pallas import tpu as pltpu
from jax.experimental.pallas import tpu_sc as plsc

N, E, G = 10000, 320000, 64
NEG = 0.01

NC, NS = 2, 16
NW = NC * NS
CH = 128
EPW = 10240
NCH = EPW // CH
EPAD = NW * EPW
NPAD = 10240
RPT = NPAD // NS

_MESH = plsc.VectorSubcoreMesh(core_axis_name="c", subcore_axis_name="s")
_SC_PARAMS = pltpu.CompilerParams(use_tc_tiling_on_sc=False)


def _leaky(v):
    return jnp.maximum(v, 0.0) + NEG * jnp.minimum(v, 0.0)


@functools.partial(
    pl.kernel,
    out_type=jax.ShapeDtypeStruct((NC, NPAD, 16), jnp.float32),
    mesh=_MESH,
    compiler_params=_SC_PARAMS,
    scratch_types=[
        pltpu.VMEM((NCH, CH), jnp.int32),
        pltpu.VMEM((CH, 16), jnp.float32),
        pltpu.VMEM_SHARED((NPAD, 16), jnp.float32),
    ],
)
def _sc_deg(dst_hbm, ones_hbm, zeros_hbm, out_hbm, dstv, onesv, acc):
    cid = lax.axis_index("c")
    sid = lax.axis_index("s")
    wid = cid * NS + sid
    pltpu.sync_copy(dst_hbm.at[pl.ds(wid * NCH, NCH)], dstv)
    pltpu.sync_copy(ones_hbm, onesv)
    r0 = sid * RPT
    pltpu.sync_copy(zeros_hbm.at[pl.ds(r0, RPT)], acc.at[pl.ds(r0, RPT)])
    plsc.subcore_barrier()

    @pl.loop(0, NCH)
    def _(j):
        pltpu.sync_copy(onesv, acc.at[dstv.at[j]], add=True)

    plsc.subcore_barrier()
    pltpu.sync_copy(acc.at[pl.ds(r0, RPT)], out_hbm.at[cid, pl.ds(r0, RPT)])


def _make_sc_agg(D):
    @functools.partial(
        pl.kernel,
        out_type=jax.ShapeDtypeStruct((NC, NPAD, D), jnp.float32),
        mesh=_MESH,
        compiler_params=_SC_PARAMS,
        scratch_types=[
            pltpu.VMEM((NCH, CH), jnp.int32),
            pltpu.VMEM((NCH, CH), jnp.int32),
            pltpu.VMEM((CH, D), jnp.float32),
            pltpu.VMEM_SHARED((NPAD, D), jnp.float32),
            pltpu.SemaphoreType.DMA,
        ],
    )
    def agg(src_hbm, dst_hbm, h_hbm, zeros_hbm, out_hbm, srcv, dstv, buf, acc, gsem):
        cid = lax.axis_index("c")
        sid = lax.axis_index("s")
        wid = cid * NS + sid
        pltpu.sync_copy(src_hbm.at[pl.ds(wid * NCH, NCH)], srcv)
        pltpu.sync_copy(dst_hbm.at[pl.ds(wid * NCH, NCH)], dstv)
        r0 = sid * RPT
        pltpu.sync_copy(zeros_hbm.at[pl.ds(r0, RPT)], acc.at[pl.ds(r0, RPT)])
        plsc.subcore_barrier()

        @pl.loop(0, NCH)
        def _(j):
            pltpu.async_copy(h_hbm.at[srcv.at[j]], buf, gsem).wait()
            pltpu.sync_copy(buf, acc.at[dstv.at[j]], add=True)

        plsc.subcore_barrier()
        pltpu.sync_copy(acc.at[pl.ds(r0, RPT)], out_hbm.at[cid, pl.ds(r0, RPT)])

    return agg


_sc_agg64 = _make_sc_agg(64)
_sc_agg16 = _make_sc_agg(16)


def _tc1_body(deg_ref, x_ref, w1_ref, ht1_ref, dis_ref):
    deg = deg_ref[0, :, 0:1] + deg_ref[1, :, 0:1] + 1.0
    dis = lax.rsqrt(deg)
    dis_ref[...] = dis
    h = jnp.dot(x_ref[...], w1_ref[...], preferred_element_type=jnp.float32)
    ht1_ref[...] = h * dis[:N]


def _tc1(deg, x, W1):
    return pl.pallas_call(
        _tc1_body,
        out_shape=(
            jax.ShapeDtypeStruct((N, 64), jnp.float32),
            jax.ShapeDtypeStruct((NPAD, 1), jnp.float32),
        ),
    )(deg, x, W1)


def _tc2_body(s1_ref, ht1_ref, dis_ref, b1_ref, w2_ref, ht2_ref):
    dn = dis_ref[:N]
    s = s1_ref[0, :N, :] + s1_ref[1, :N, :] + ht1_ref[...]
    agg1 = dn * s + b1_ref[...]
    h2 = _leaky(agg1)
    ht2_ref[...] = jnp.dot(h2, w2_ref[...], preferred_element_type=jnp.float32) * dn


def _tc2(s1, ht1, dis, b1r, W2p):
    return pl.pallas_call(
        _tc2_body,
        out_shape=jax.ShapeDtypeStruct((N, 16), jnp.float32),
    )(s1, ht1, dis, b1r, W2p)


def _tc3_body(s2_ref, ht2_ref, dis_ref, b2_ref, batch_ref, gf_ref,
              fw1_ref, fb1_ref, fw2_ref, fb2_ref, out_ref):
    dn = dis_ref[:N]
    s = s2_ref[0, :N, :] + s2_ref[1, :N, :] + ht2_ref[...]
    hf = _leaky(dn * s + b2_ref[...])
    gids = lax.broadcasted_iota(jnp.int32, (1, G), 1)
    onehot = (batch_ref[...] == gids).astype(jnp.float32)
    sums = lax.dot_general(onehot, hf, (((0,), (0,)), ((), ())),
                           preferred_element_type=jnp.float32)
    cnts = jnp.sum(onehot, axis=0)[:, None]
    pool = sums[:, :8] / jnp.maximum(cnts, 1.0)
    cat = jnp.concatenate([pool, gf_ref[...]], axis=1)
    z = _leaky(jnp.dot(cat, fw1_ref[...], preferred_element_type=jnp.float32)
               + fb1_ref[...])
    out_ref[...] = _leaky(jnp.dot(z, fw2_ref[...], preferred_element_type=jnp.float32)
                          + fb2_ref[...])


def _tc3(s2, ht2, dis, b2r, batch_col, gf, fW1, fb1r, fW2, fb2r):
    return pl.pallas_call(
        _tc3_body,
        out_shape=jax.ShapeDtypeStruct((G, 8), jnp.float32),
    )(s2, ht2, dis, b2r, batch_col, gf, fW1, fb1r, fW2, fb2r)


def kernel(x, edge_index, global_features, batch, W1, b1, W2, b2, fW1, fb1, fW2, fb2):
    src = edge_index[0].astype(jnp.int32)
    dst = edge_index[1].astype(jnp.int32)
    pad = EPAD - E
    src2d = jnp.concatenate([src, jnp.zeros((pad,), jnp.int32)]).reshape(EPAD // CH, CH)
    dst2d = jnp.concatenate([dst, jnp.full((pad,), N, jnp.int32)]).reshape(EPAD // CH, CH)
    batch_col = batch.astype(jnp.int32).reshape(N, 1)

    ones16 = jnp.ones((CH, 16), jnp.float32)
    zeros16 = jnp.zeros((NPAD, 16), jnp.float32)
    zeros64 = jnp.zeros((NPAD, 64), jnp.float32)

    deg = _sc_deg(dst2d, ones16, zeros16)
    ht1, dis = _tc1(deg, x, W1)
    s1 = _sc_agg64(src2d, dst2d, ht1, zeros64)
    W2p = jnp.concatenate([W2, jnp.zeros((64, 8), jnp.float32)], axis=1)
    ht2 = _tc2(s1, ht1, dis, b1.reshape(1, 64), W2p)
    s2 = _sc_agg16(src2d, dst2d, ht2, zeros16)
    b2p = jnp.concatenate([b2, jnp.zeros((8,), jnp.float32)]).reshape(1, 16)
    return _tc3(s2, ht2, dis, b2p, batch_col, global_features,
                fW1, fb1.reshape(1, 16), fW2, fb2.reshape(1, 8))


# speedup vs baseline: 25.9355x; 1.2326x over previous
import functools

import jax
import jax.numpy as jnp
from jax import lax
from jax.experimental import pallas as pl
from jax.experimental.pallas import tpu as pltpu
from jax.experimental.pallas import tpu_sc as plsc

N, E, G = 10000, 320000, 64
NEG = 0.01

NC, NS = 2, 16
NW = NC * NS
CH = 128
EPW = 10240
NCH = EPW // CH
EPAD = NW * EPW
NPAD = 10240
RPT = NPAD // NS

_MESH = plsc.VectorSubcoreMesh(core_axis_name="c", subcore_axis_name="s")
_SC_PARAMS = pltpu.CompilerParams(use_tc_tiling_on_sc=False)


def _leaky(v):
    return jnp.maximum(v, 0.0) + NEG * jnp.minimum(v, 0.0)


@functools.partial(
    pl.kernel,
    out_type=jax.ShapeDtypeStruct((NC, NPAD, 16), jnp.float32),
    mesh=_MESH,
    compiler_params=_SC_PARAMS,
    scratch_types=[
        pltpu.VMEM((NCH, CH), jnp.int32),
        pltpu.VMEM((CH, 16), jnp.float32),
        pltpu.VMEM_SHARED((NPAD, 16), jnp.float32),
        pltpu.SemaphoreType.DMA,
    ],
)
def _sc_deg(dst_hbm, ones_hbm, zeros_hbm, out_hbm, dstv, onesv, acc, ssem):
    cid = lax.axis_index("c")
    sid = lax.axis_index("s")
    wid = cid * NS + sid
    pltpu.sync_copy(dst_hbm.at[pl.ds(wid * NCH, NCH)], dstv)
    pltpu.sync_copy(ones_hbm, onesv)
    r0 = sid * RPT
    pltpu.sync_copy(zeros_hbm.at[pl.ds(r0, RPT)], acc.at[pl.ds(r0, RPT)])
    plsc.subcore_barrier()

    @pl.loop(0, NCH)
    def _(j):
        pltpu.async_copy(onesv, acc.at[dstv.at[j]], ssem, add=True)

    @pl.loop(0, NCH)
    def _(j):
        pltpu.make_async_copy(onesv, acc.at[dstv.at[j]], ssem).wait()

    plsc.subcore_barrier()
    pltpu.sync_copy(acc.at[pl.ds(r0, RPT)], out_hbm.at[cid, pl.ds(r0, RPT)])


NB = 8
NG = NCH // NB


def _make_sc_agg(D):
    @functools.partial(
        pl.kernel,
        out_type=jax.ShapeDtypeStruct((NC, NPAD, D), jnp.float32),
        mesh=_MESH,
        compiler_params=_SC_PARAMS,
        scratch_types=[
            pltpu.VMEM((NCH, CH), jnp.int32),
            pltpu.VMEM((NCH, CH), jnp.int32),
            pltpu.VMEM((NB, CH, D), jnp.float32),
            pltpu.VMEM_SHARED((NPAD, D), jnp.float32),
            pltpu.SemaphoreType.DMA((NB,)),
            pltpu.SemaphoreType.DMA((NB,)),
        ],
    )
    def agg(src_hbm, dst_hbm, h_hbm, zeros_hbm, out_hbm, srcv, dstv, buf, acc,
            gsem, ssem):
        cid = lax.axis_index("c")
        sid = lax.axis_index("s")
        wid = cid * NS + sid
        pltpu.sync_copy(src_hbm.at[pl.ds(wid * NCH, NCH)], srcv)
        pltpu.sync_copy(dst_hbm.at[pl.ds(wid * NCH, NCH)], dstv)
        r0 = sid * RPT
        pltpu.sync_copy(zeros_hbm.at[pl.ds(r0, RPT)], acc.at[pl.ds(r0, RPT)])
        plsc.subcore_barrier()

        @pl.loop(0, NG)
        def _(i):
            for b in range(NB):
                j = i * NB + b

                @pl.when(i > 0)
                def _():
                    pltpu.make_async_copy(
                        buf.at[b], acc.at[dstv.at[j - NB]], ssem.at[b]).wait()

                pltpu.async_copy(h_hbm.at[srcv.at[j]], buf.at[b], gsem.at[b])
            for b in range(NB):
                j = i * NB + b
                pltpu.make_async_copy(
                    h_hbm.at[srcv.at[j]], buf.at[b], gsem.at[b]).wait()
                pltpu.async_copy(
                    buf.at[b], acc.at[dstv.at[j]], ssem.at[b], add=True)

        for b in range(NB):
            j = (NG - 1) * NB + b
            pltpu.make_async_copy(
                buf.at[b], acc.at[dstv.at[j]], ssem.at[b]).wait()

        plsc.subcore_barrier()
        pltpu.sync_copy(acc.at[pl.ds(r0, RPT)], out_hbm.at[cid, pl.ds(r0, RPT)])

    return agg


_sc_agg64 = _make_sc_agg(64)
_sc_agg16 = _make_sc_agg(16)


def _tc1_body(deg_ref, x_ref, w1_ref, ht1_ref, dis_ref):
    deg = deg_ref[0, :, 0:1] + deg_ref[1, :, 0:1] + 1.0
    dis = lax.rsqrt(deg)
    dis_ref[...] = dis
    h = jnp.dot(x_ref[...], w1_ref[...], preferred_element_type=jnp.float32)
    ht1_ref[...] = h * dis[:N]


def _tc1(deg, x, W1):
    return pl.pallas_call(
        _tc1_body,
        out_shape=(
            jax.ShapeDtypeStruct((N, 64), jnp.float32),
            jax.ShapeDtypeStruct((NPAD, 1), jnp.float32),
        ),
    )(deg, x, W1)


def _tc2_body(s1_ref, ht1_ref, dis_ref, b1_ref, w2_ref, ht2_ref):
    dn = dis_ref[:N]
    s = s1_ref[0, :N, :] + s1_ref[1, :N, :] + ht1_ref[...]
    agg1 = dn * s + b1_ref[...]
    h2 = _leaky(agg1)
    ht2_ref[...] = jnp.dot(h2, w2_ref[...], preferred_element_type=jnp.float32) * dn


def _tc2(s1, ht1, dis, b1r, W2p):
    return pl.pallas_call(
        _tc2_body,
        out_shape=jax.ShapeDtypeStruct((N, 16), jnp.float32),
    )(s1, ht1, dis, b1r, W2p)


def _tc3_body(s2_ref, ht2_ref, dis_ref, b2_ref, batch_ref, gf_ref,
              fw1_ref, fb1_ref, fw2_ref, fb2_ref, out_ref):
    dn = dis_ref[:N]
    s = s2_ref[0, :N, :] + s2_ref[1, :N, :] + ht2_ref[...]
    hf = _leaky(dn * s + b2_ref[...])
    gids = lax.broadcasted_iota(jnp.int32, (1, G), 1)
    onehot = (batch_ref[...] == gids).astype(jnp.float32)
    sums = lax.dot_general(onehot, hf, (((0,), (0,)), ((), ())),
                           preferred_element_type=jnp.float32)
    cnts = jnp.sum(onehot, axis=0)[:, None]
    pool = sums[:, :8] / jnp.maximum(cnts, 1.0)
    cat = jnp.concatenate([pool, gf_ref[...]], axis=1)
    z = _leaky(jnp.dot(cat, fw1_ref[...], preferred_element_type=jnp.float32)
               + fb1_ref[...])
    out_ref[...] = _leaky(jnp.dot(z, fw2_ref[...], preferred_element_type=jnp.float32)
                          + fb2_ref[...])


def _tc3(s2, ht2, dis, b2r, batch_col, gf, fW1, fb1r, fW2, fb2r):
    return pl.pallas_call(
        _tc3_body,
        out_shape=jax.ShapeDtypeStruct((G, 8), jnp.float32),
    )(s2, ht2, dis, b2r, batch_col, gf, fW1, fb1r, fW2, fb2r)


def kernel(x, edge_index, global_features, batch, W1, b1, W2, b2, fW1, fb1, fW2, fb2):
    src = edge_index[0].astype(jnp.int32)
    dst = edge_index[1].astype(jnp.int32)
    pad = EPAD - E
    src2d = jnp.concatenate([src, jnp.zeros((pad,), jnp.int32)]).reshape(EPAD // CH, CH)
    dst2d = jnp.concatenate([dst, jnp.full((pad,), N, jnp.int32)]).reshape(EPAD // CH, CH)
    batch_col = batch.astype(jnp.int32).reshape(N, 1)

    ones16 = jnp.ones((CH, 16), jnp.float32)
    zeros16 = jnp.zeros((NPAD, 16), jnp.float32)
    zeros64 = jnp.zeros((NPAD, 64), jnp.float32)

    deg = _sc_deg(dst2d, ones16, zeros16)
    ht1, dis = _tc1(deg, x, W1)
    s1 = _sc_agg64(src2d, dst2d, ht1, zeros64)
    W2p = jnp.concatenate([W2, jnp.zeros((64, 8), jnp.float32)], axis=1)
    ht2 = _tc2(s1, ht1, dis, b1.reshape(1, 64), W2p)
    s2 = _sc_agg16(src2d, dst2d, ht2, zeros16)
    b2p = jnp.concatenate([b2, jnp.zeros((8,), jnp.float32)]).reshape(1, 16)
    return _tc3(s2, ht2, dis, b2p, batch_col, global_features,
                fW1, fb1.reshape(1, 16), fW2, fb2.reshape(1, 8))


# speedup vs baseline: 39.7959x; 1.5344x over previous
import functools

import jax
import jax.numpy as jnp
from jax import lax
from jax.experimental import pallas as pl
from jax.experimental.pallas import tpu as pltpu
from jax.experimental.pallas import tpu_sc as plsc

N, E, G = 10000, 320000, 64
NEG = 0.01

NC, NS = 2, 16
NW = NC * NS
CH = 128
EPW = 10240
NCH = EPW // CH
EPAD = NW * EPW
TCH = EPAD // CH
NCH2 = TCH // NS
NPAD = 10240
RPT = NPAD // NS
HPT = N // NS

NB = 8

_MESH = plsc.VectorSubcoreMesh(core_axis_name="c", subcore_axis_name="s")
_SC_PARAMS = pltpu.CompilerParams(use_tc_tiling_on_sc=False)


def _leaky(v):
    return jnp.maximum(v, 0.0) + NEG * jnp.minimum(v, 0.0)


def _zero_acc(zrow_hbm, zrowv, acc, r0):
    pltpu.sync_copy(zrow_hbm, zrowv)
    for t in range(RPT // CH):
        pltpu.sync_copy(zrowv, acc.at[pl.ds(r0 + t * CH, CH)])


@functools.partial(
    pl.kernel,
    out_type=jax.ShapeDtypeStruct((NC, NPAD, 16), jnp.float32),
    mesh=_MESH,
    compiler_params=_SC_PARAMS,
    scratch_types=[
        pltpu.VMEM((NCH, CH), jnp.int32),
        pltpu.VMEM((CH, 16), jnp.float32),
        pltpu.VMEM((CH, 16), jnp.float32),
        pltpu.VMEM_SHARED((NPAD, 16), jnp.float32),
        pltpu.SemaphoreType.DMA,
    ],
)
def _sc_deg(dst_hbm, ones_hbm, zrow_hbm, out_hbm, dstv, onesv, zrowv, acc, ssem):
    cid = lax.axis_index("c")
    sid = lax.axis_index("s")
    wid = cid * NS + sid
    pltpu.sync_copy(dst_hbm.at[pl.ds(wid * NCH, NCH)], dstv)
    pltpu.sync_copy(ones_hbm, onesv)
    r0 = sid * RPT
    _zero_acc(zrow_hbm, zrowv, acc, r0)
    plsc.subcore_barrier()

    @pl.loop(0, NCH)
    def _(j):
        pltpu.async_copy(onesv, acc.at[dstv.at[j]], ssem, add=True)

    @pl.loop(0, NCH)
    def _(j):
        pltpu.make_async_copy(onesv, acc.at[dstv.at[j]], ssem).wait()

    plsc.subcore_barrier()
    pltpu.sync_copy(acc.at[pl.ds(r0, RPT)], out_hbm.at[cid, pl.ds(r0, RPT)])


def _agg_loop(h_sh, srcv, dstv, buf, acc, gsem, ssem, nch):
    ng = nch // NB

    @pl.loop(0, ng)
    def _(i):
        for b in range(NB):
            j = i * NB + b

            @pl.when(i > 0)
            def _():
                pltpu.make_async_copy(
                    buf.at[b], acc.at[dstv.at[j - NB]], ssem.at[b]).wait()

            pltpu.async_copy(h_sh.at[srcv.at[j]], buf.at[b], gsem.at[b])
        for b in range(NB):
            j = i * NB + b
            pltpu.make_async_copy(
                h_sh.at[srcv.at[j]], buf.at[b], gsem.at[b]).wait()
            pltpu.async_copy(
                buf.at[b], acc.at[dstv.at[j]], ssem.at[b], add=True)

    for b in range(NB):
        j = (ng - 1) * NB + b
        pltpu.make_async_copy(
            buf.at[b], acc.at[dstv.at[j]], ssem.at[b]).wait()


D1 = 32


@functools.partial(
    pl.kernel,
    out_type=jax.ShapeDtypeStruct((NC, NPAD, D1), jnp.float32),
    mesh=_MESH,
    compiler_params=_SC_PARAMS,
    scratch_types=[
        pltpu.VMEM((NCH2, CH), jnp.int32),
        pltpu.VMEM((NCH2, CH), jnp.int32),
        pltpu.VMEM((NB, CH, D1), jnp.float32),
        pltpu.VMEM((CH, D1), jnp.float32),
        pltpu.VMEM_SHARED((NPAD, D1), jnp.float32),
        pltpu.VMEM_SHARED((N, D1), jnp.float32),
        pltpu.SemaphoreType.DMA((NB,)),
        pltpu.SemaphoreType.DMA((NB,)),
    ],
)
def _sc_agg1(src_hbm, dst_hbm, h_hbm, zrow_hbm, out_hbm,
             srcv, dstv, buf, zrowv, acc, h_sh, gsem, ssem):
    cid = lax.axis_index("c")
    sid = lax.axis_index("s")
    pltpu.sync_copy(src_hbm.at[pl.ds(sid * NCH2, NCH2)], srcv)
    pltpu.sync_copy(dst_hbm.at[pl.ds(sid * NCH2, NCH2)], dstv)
    r0 = sid * RPT
    _zero_acc(zrow_hbm, zrowv, acc, r0)
    pltpu.sync_copy(h_hbm.at[cid, pl.ds(sid * HPT, HPT)],
                    h_sh.at[pl.ds(sid * HPT, HPT)])
    plsc.subcore_barrier()
    _agg_loop(h_sh, srcv, dstv, buf, acc, gsem, ssem, NCH2)
    plsc.subcore_barrier()
    pltpu.sync_copy(acc.at[pl.ds(r0, RPT)], out_hbm.at[cid, pl.ds(r0, RPT)])


D2 = 16


@functools.partial(
    pl.kernel,
    out_type=jax.ShapeDtypeStruct((NC, NPAD, D2), jnp.float32),
    mesh=_MESH,
    compiler_params=_SC_PARAMS,
    scratch_types=[
        pltpu.VMEM((NCH, CH), jnp.int32),
        pltpu.VMEM((NCH, CH), jnp.int32),
        pltpu.VMEM((NB, CH, D2), jnp.float32),
        pltpu.VMEM((CH, D2), jnp.float32),
        pltpu.VMEM_SHARED((NPAD, D2), jnp.float32),
        pltpu.VMEM_SHARED((N, D2), jnp.float32),
        pltpu.SemaphoreType.DMA((NB,)),
        pltpu.SemaphoreType.DMA((NB,)),
    ],
)
def _sc_agg2(src_hbm, dst_hbm, h_hbm, zrow_hbm, out_hbm,
             srcv, dstv, buf, zrowv, acc, h_sh, gsem, ssem):
    cid = lax.axis_index("c")
    sid = lax.axis_index("s")
    wid = cid * NS + sid
    pltpu.sync_copy(src_hbm.at[pl.ds(wid * NCH, NCH)], srcv)
    pltpu.sync_copy(dst_hbm.at[pl.ds(wid * NCH, NCH)], dstv)
    r0 = sid * RPT
    _zero_acc(zrow_hbm, zrowv, acc, r0)
    pltpu.sync_copy(h_hbm.at[pl.ds(sid * HPT, HPT)], h_sh.at[pl.ds(sid * HPT, HPT)])
    plsc.subcore_barrier()
    _agg_loop(h_sh, srcv, dstv, buf, acc, gsem, ssem, NCH)
    plsc.subcore_barrier()
    pltpu.sync_copy(acc.at[pl.ds(r0, RPT)], out_hbm.at[cid, pl.ds(r0, RPT)])


def _tc1_body(deg_ref, x_ref, w1_ref, ht1_ref, dis_ref):
    deg = deg_ref[0, :, 0:1] + deg_ref[1, :, 0:1] + 1.0
    dis = lax.rsqrt(deg)
    dis_ref[...] = dis
    h = jnp.dot(x_ref[...], w1_ref[...], preferred_element_type=jnp.float32)
    h = h * dis[:N]
    ht1_ref[0] = h[:, :D1]
    ht1_ref[1] = h[:, D1:]


def _tc1(deg, x, W1):
    return pl.pallas_call(
        _tc1_body,
        out_shape=(
            jax.ShapeDtypeStruct((NC, N, D1), jnp.float32),
            jax.ShapeDtypeStruct((NPAD, 1), jnp.float32),
        ),
    )(deg, x, W1)


def _tc2_body(s1_ref, ht1_ref, dis_ref, b1_ref, w2_ref, ht2_ref):
    dn = dis_ref[:N]
    s = jnp.concatenate([s1_ref[0, :N, :] + ht1_ref[0],
                         s1_ref[1, :N, :] + ht1_ref[1]], axis=1)
    agg1 = dn * s + b1_ref[...]
    h2 = _leaky(agg1)
    ht2_ref[...] = jnp.dot(h2, w2_ref[...], preferred_element_type=jnp.float32) * dn


def _tc2(s1, ht1, dis, b1r, W2p):
    return pl.pallas_call(
        _tc2_body,
        out_shape=jax.ShapeDtypeStruct((N, D2), jnp.float32),
    )(s1, ht1, dis, b1r, W2p)


def _tc3_body(s2_ref, ht2_ref, dis_ref, b2_ref, batch_ref, gf_ref,
              fw1_ref, fb1_ref, fw2_ref, fb2_ref, out_ref):
    dn = dis_ref[:N]
    s = s2_ref[0, :N, :] + s2_ref[1, :N, :] + ht2_ref[...]
    hf = _leaky(dn * s + b2_ref[...])
    gids = lax.broadcasted_iota(jnp.int32, (1, G), 1)
    onehot = (batch_ref[...] == gids).astype(jnp.float32)
    sums = lax.dot_general(onehot, hf, (((0,), (0,)), ((), ())),
                           preferred_element_type=jnp.float32)
    cnts = jnp.sum(onehot, axis=0)[:, None]
    pool = sums[:, :8] / jnp.maximum(cnts, 1.0)
    cat = jnp.concatenate([pool, gf_ref[...]], axis=1)
    z = _leaky(jnp.dot(cat, fw1_ref[...], preferred_element_type=jnp.float32)
               + fb1_ref[...])
    out_ref[...] = _leaky(jnp.dot(z, fw2_ref[...], preferred_element_type=jnp.float32)
                          + fb2_ref[...])


def _tc3(s2, ht2, dis, b2r, batch_col, gf, fW1, fb1r, fW2, fb2r):
    return pl.pallas_call(
        _tc3_body,
        out_shape=jax.ShapeDtypeStruct((G, 8), jnp.float32),
    )(s2, ht2, dis, b2r, batch_col, gf, fW1, fb1r, fW2, fb2r)


def kernel(x, edge_index, global_features, batch, W1, b1, W2, b2, fW1, fb1, fW2, fb2):
    src = edge_index[0].astype(jnp.int32)
    dst = edge_index[1].astype(jnp.int32)
    pad = EPAD - E
    src2d = jnp.concatenate([src, jnp.zeros((pad,), jnp.int32)]).reshape(TCH, CH)
    dst2d = jnp.concatenate([dst, jnp.full((pad,), N, jnp.int32)]).reshape(TCH, CH)
    batch_col = batch.astype(jnp.int32).reshape(N, 1)

    ones16 = jnp.ones((CH, 16), jnp.float32)
    zrow16 = jnp.zeros((CH, 16), jnp.float32)
    zrow32 = jnp.zeros((CH, D1), jnp.float32)

    deg = _sc_deg(dst2d, ones16, zrow16)
    ht1, dis = _tc1(deg, x, W1)
    s1 = _sc_agg1(src2d, dst2d, ht1, zrow32)
    W2p = jnp.concatenate([W2, jnp.zeros((64, 8), jnp.float32)], axis=1)
    ht2 = _tc2(s1, ht1, dis, b1.reshape(1, 64), W2p)
    s2 = _sc_agg2(src2d, dst2d, ht2, zrow16)
    b2p = jnp.concatenate([b2, jnp.zeros((8,), jnp.float32)]).reshape(1, 16)
    return _tc3(s2, ht2, dis, b2p, batch_col, global_features,
                fW1, fb1.reshape(1, 16), fW2, fb2.reshape(1, 8))
